# baseline (speedup 1.0000x reference)
import functools

import jax
import jax.numpy as jnp
from jax import lax
from jax.experimental import pallas as pl
from jax.experimental.pallas import tpu as pltpu
from jax.experimental.pallas import tpu_sc as plsc

B = 4096
L = 26
VOCAB = 100000
EMB = 64
U = 128

NC = 2
NS = 16
NW = NC * NS
N = B * L
PER_W = N // NW
K = 128
NCHUNK = PER_W // K
BPW = B // NW


V1P = 100096


def _sc_gather(idxb3, bidx2, table2, t1_flat):
    mesh = plsc.VectorSubcoreMesh(core_axis_name="c", subcore_axis_name="s")

    @functools.partial(
        pl.kernel,
        out_type=(
            jax.ShapeDtypeStruct((N // 2, 2 * EMB), jnp.float32),
            jax.ShapeDtypeStruct((NW, NCHUNK, K), jnp.float32),
            jax.ShapeDtypeStruct((NW, BPW), jnp.float32),
        ),
        mesh=mesh,
        compiler_params=pltpu.CompilerParams(
            use_tc_tiling_on_sc=False, needs_layout_passes=False),
        scratch_types=[
            pltpu.VMEM((PER_W,), jnp.int32),
            pltpu.VMEM((2, K), jnp.int32),
            pltpu.VMEM((2, K, EMB), jnp.float32),
            pltpu.VMEM((NCHUNK, K), jnp.float32),
            pltpu.VMEM((BPW,), jnp.int32),
            pltpu.VMEM((BPW,), jnp.float32),
            pltpu.VMEM((V1P,), jnp.float32),
            pltpu.SemaphoreType.DMA,
            pltpu.SemaphoreType.DMA,
            pltpu.SemaphoreType.DMA,
        ],
    )
    def k(idxb_hbm, bidx_hbm, t2_hbm, t1_hbm, emb_hbm, one_hbm,
          bias_hbm, idxb_v, cidx_v, rows_v, one_v, bidx_v, brow_v, tab_v,
          sem0, sem1, semt):
        wid = lax.axis_index("s") * NC + lax.axis_index("c")
        dbase = wid * K

        pltpu.sync_copy(idxb_hbm.at[wid], idxb_v)
        pltpu.sync_copy(bidx_hbm.at[wid], bidx_v)
        tcopy = pltpu.async_copy(t1_hbm, tab_v, semt)

        def build(l, buf):
            for kk in range(K // 16):
                pos = (jnp.arange(16, dtype=jnp.int32) + (kk * 16)) * L + l
                cidx_v[buf, pl.ds(kk * 16, 16)] = plsc.load_gather(
                    idxb_v, [pos])

        def fire(l, buf, sem):
            pltpu.async_copy(t2_hbm.at[cidx_v.at[buf]], rows_v.at[buf], sem)

        def drain(buf, sem):
            pltpu.make_async_copy(
                t2_hbm.at[cidx_v.at[0]], rows_v.at[buf], sem).wait()

        def write(l, buf):
            pltpu.sync_copy(
                rows_v.at[buf],
                emb_hbm.at[pl.ds((l // 2) * B + dbase, K),
                           pl.ds((l % 2) * EMB, EMB)])

        build(0, 0)
        fire(0, 0, sem0)

        def pair(p, carry):
            c0 = 2 * p
            build(c0 + 1, 1)
            fire(c0 + 1, 1, sem1)
            drain(0, sem0)
            write(c0, 0)

            @pl.when(c0 + 2 < NCHUNK)
            def _():
                build(c0 + 2, 0)
                fire(c0 + 2, 0, sem0)

            drain(1, sem1)
            write(c0 + 1, 1)
            return carry

        lax.fori_loop(0, NCHUNK // 2, pair, 0)

        tcopy.wait()
        for i in range(BPW // 16):
            brow_v[pl.ds(i * 16, 16)] = plsc.load_gather(
                tab_v, [bidx_v[pl.ds(i * 16, 16)]])
        pltpu.sync_copy(brow_v, bias_hbm.at[wid])

        def one_body(j, carry):
            for i in range(K // 16):
                one_v[j, pl.ds(i * 16, 16)] = plsc.load_gather(
                    tab_v, [idxb_v[pl.ds(j * K + i * 16, 16)]])
            return carry

        lax.fori_loop(0, NCHUNK, one_body, 0)
        pltpu.sync_copy(one_v, one_hbm.at[wid])

    return k(idxb3, bidx2, table2, t1_flat)


def _tc_body(emb_ref, vals_ref, one_ref, bias_ref, wd_ref, bd_ref,
             wdnn_ref, bdnn_ref, wout_ref, bout_ref, out_ref):
    bt = vals_ref.shape[0]
    wd = wd_ref[...]
    bd = bd_ref[...]
    sum_v = jnp.zeros((bt, EMB), jnp.float32)
    sum_sq = jnp.zeros((bt, EMB), jnp.float32)
    accd = jnp.zeros((bt, 1), jnp.float32)
    for l in range(L):
        e = emb_ref[l // 2][:, (l % 2) * EMB:(l % 2) * EMB + EMB]
        h = jnp.dot(e, wd, preferred_element_type=jnp.float32) + bd
        h = jnp.maximum(h, 0.0)
        accd += jnp.sum(h * wdnn_ref[l], axis=1, keepdims=True)
        v = e * vals_ref[:, l:l + 1]
        sum_v += v
        sum_sq += v * v
    y_fm1 = jnp.sum(vals_ref[...] * one_ref[...], axis=1, keepdims=True) + bias_ref[...]
    y_fm2 = 0.5 * jnp.sum(sum_v * sum_v - sum_sq, axis=1, keepdims=True)
    y_dnn = jnp.maximum(accd + bdnn_ref[...], 0.0)
    out_ref[...] = (y_fm1 * wout_ref[0:1, 0:1] + y_fm2 * wout_ref[0:1, 1:2]
                    + y_dnn * wout_ref[0:1, 2:3] + bout_ref[...])


BT = 1024


def _tc_compute(emb3, vals, one_bl, bias2, W_dense, b_dense, W_dnn_r, b_dnn,
                W_out_r, b_out):
    grid = (B // BT,)
    return pl.pallas_call(
        _tc_body,
        grid=grid,
        in_specs=[
            pl.BlockSpec((L // 2, BT, 2 * EMB), lambda i: (0, i, 0)),
            pl.BlockSpec((BT, L), lambda i: (i, 0)),
            pl.BlockSpec((BT, L), lambda i: (i, 0)),
            pl.BlockSpec((BT, 1), lambda i: (i, 0)),
            pl.BlockSpec((EMB, U), lambda i: (0, 0)),
            pl.BlockSpec((1, U), lambda i: (0, 0)),
            pl.BlockSpec((L, U), lambda i: (0, 0)),
            pl.BlockSpec((1, 1), lambda i: (0, 0)),
            pl.BlockSpec((1, 3), lambda i: (0, 0)),
            pl.BlockSpec((1, 1), lambda i: (0, 0)),
        ],
        out_specs=pl.BlockSpec((BT, 1), lambda i: (i, 0)),
        out_shape=jax.ShapeDtypeStruct((B, 1), jnp.float32),
    )(emb3, vals, one_bl, bias2, W_dense, b_dense, W_dnn_r, b_dnn, W_out_r, b_out)


def kernel(input_values, input_indexes, bias_indexes, one_order_table,
           two_order_table, W_dense, b_dense, W_dnn, b_dnn, W_out, b_out):
    idxb3 = input_indexes.astype(jnp.int32).reshape(NW, PER_W)
    bidx2 = bias_indexes.astype(jnp.int32).reshape(NW, BPW)
    t1_flat = jnp.concatenate(
        [one_order_table[:, 0],
         jnp.zeros((V1P - VOCAB - 1,), jnp.float32)])
    emb, onev, biasv = _sc_gather(idxb3, bidx2, two_order_table, t1_flat)
    emb3 = emb.reshape(L // 2, B, 2 * EMB)
    one_bl = onev.reshape(B, L)
    bias2 = biasv.reshape(B, 1)
    return _tc_compute(
        emb3, input_values, one_bl, bias2,
        W_dense, b_dense.reshape(1, U), W_dnn.reshape(L, U),
        b_dnn.reshape(1, 1), W_out.reshape(1, 3), b_out.reshape(1, 1))


# speedup vs baseline: 1.1081x; 1.1081x over previous
import functools

import jax
import jax.numpy as jnp
from jax import lax
from jax.experimental import pallas as pl
from jax.experimental.pallas import tpu as pltpu
from jax.experimental.pallas import tpu_sc as plsc

B = 4096
L = 26
VOCAB = 100000
EMB = 64
U = 128

NC = 2
NS = 16
NW = NC * NS
N = B * L
PER_W = N // NW
K = 128
NCHUNK = PER_W // K
BPW = B // NW


V1P = 100096


def _sc_gather(idxb3, bidx2, table2, t1_flat):
    mesh = plsc.VectorSubcoreMesh(core_axis_name="c", subcore_axis_name="s")

    @functools.partial(
        pl.kernel,
        out_type=(
            jax.ShapeDtypeStruct((N, 2 * EMB), jnp.float32),
            jax.ShapeDtypeStruct((NW, NCHUNK, K), jnp.float32),
            jax.ShapeDtypeStruct((NW, BPW), jnp.float32),
        ),
        mesh=mesh,
        compiler_params=pltpu.CompilerParams(
            use_tc_tiling_on_sc=False, needs_layout_passes=False),
        scratch_types=[
            pltpu.VMEM((PER_W,), jnp.int32),
            pltpu.VMEM((2, K), jnp.int32),
            pltpu.VMEM((2, K, EMB), jnp.float32),
            pltpu.VMEM((NCHUNK, K), jnp.float32),
            pltpu.VMEM((BPW,), jnp.int32),
            pltpu.VMEM((BPW,), jnp.float32),
            pltpu.VMEM((V1P,), jnp.float32),
            pltpu.SemaphoreType.DMA,
            pltpu.SemaphoreType.DMA,
            pltpu.SemaphoreType.DMA,
        ],
    )
    def k(idxb_hbm, bidx_hbm, t2_hbm, t1_hbm, emb_hbm, one_hbm,
          bias_hbm, idxb_v, cidx_v, rows_v, one_v, bidx_v, brow_v, tab_v,
          sem0, sem1, semt):
        wid = lax.axis_index("s") * NC + lax.axis_index("c")
        dbase = wid * K

        pltpu.sync_copy(idxb_hbm.at[wid], idxb_v)
        pltpu.sync_copy(bidx_hbm.at[wid], bidx_v)
        tcopy = pltpu.async_copy(t1_hbm, tab_v, semt)

        def build(l, buf):
            for kk in range(K // 16):
                pos = (jnp.arange(16, dtype=jnp.int32) + (kk * 16)) * L + l
                cidx_v[buf, pl.ds(kk * 16, 16)] = plsc.load_gather(
                    idxb_v, [pos])

        def fire(l, buf, sem):
            pltpu.async_copy(t2_hbm.at[cidx_v.at[buf]], rows_v.at[buf], sem)

        def drain(buf, sem):
            pltpu.make_async_copy(
                t2_hbm.at[cidx_v.at[0]], rows_v.at[buf], sem).wait()

        def write(l, buf):
            pltpu.sync_copy(
                rows_v.at[buf],
                emb_hbm.at[pl.ds(l * B + dbase, K), pl.ds(0, EMB)])

        build(0, 0)
        fire(0, 0, sem0)

        def pair(p, carry):
            c0 = 2 * p
            build(c0 + 1, 1)
            fire(c0 + 1, 1, sem1)
            drain(0, sem0)
            write(c0, 0)

            @pl.when(c0 + 2 < NCHUNK)
            def _():
                build(c0 + 2, 0)
                fire(c0 + 2, 0, sem0)

            drain(1, sem1)
            write(c0 + 1, 1)
            return carry

        lax.fori_loop(0, NCHUNK // 2, pair, 0)

        tcopy.wait()
        for i in range(BPW // 16):
            brow_v[pl.ds(i * 16, 16)] = plsc.load_gather(
                tab_v, [bidx_v[pl.ds(i * 16, 16)]])
        pltpu.sync_copy(brow_v, bias_hbm.at[wid])

        def one_body(j, carry):
            for i in range(K // 16):
                one_v[j, pl.ds(i * 16, 16)] = plsc.load_gather(
                    tab_v, [idxb_v[pl.ds(j * K + i * 16, 16)]])
            return carry

        lax.fori_loop(0, NCHUNK, one_body, 0)
        pltpu.sync_copy(one_v, one_hbm.at[wid])

    return k(idxb3, bidx2, table2, t1_flat)


def _tc_body(emb_ref, vals_ref, one_ref, bias_ref, wd_ref, bd_ref,
             wdnn_ref, bdnn_ref, wout_ref, bout_ref, out_ref):
    bt = vals_ref.shape[0]
    wd = wd_ref[...]
    bd = bd_ref[...]
    sum_v = jnp.zeros((bt, EMB), jnp.float32)
    sum_sq = jnp.zeros((bt, EMB), jnp.float32)
    accd = jnp.zeros((bt, 1), jnp.float32)
    for l in range(L):
        e = emb_ref[l][:, 0:EMB]
        h = jnp.dot(e, wd, preferred_element_type=jnp.float32) + bd
        h = jnp.maximum(h, 0.0)
        accd += jnp.sum(h * wdnn_ref[l], axis=1, keepdims=True)
        v = e * vals_ref[:, l:l + 1]
        sum_v += v
        sum_sq += v * v
    y_fm1 = jnp.sum(vals_ref[...] * one_ref[...], axis=1, keepdims=True) + bias_ref[...]
    y_fm2 = 0.5 * jnp.sum(sum_v * sum_v - sum_sq, axis=1, keepdims=True)
    y_dnn = jnp.maximum(accd + bdnn_ref[...], 0.0)
    out_ref[...] = (y_fm1 * wout_ref[0:1, 0:1] + y_fm2 * wout_ref[0:1, 1:2]
                    + y_dnn * wout_ref[0:1, 2:3] + bout_ref[...])


BT = 512


def _tc_compute(emb3, vals, one_bl, bias2, W_dense, b_dense, W_dnn_r, b_dnn,
                W_out_r, b_out):
    grid = (B // BT,)
    return pl.pallas_call(
        _tc_body,
        grid=grid,
        in_specs=[
            pl.BlockSpec((L, BT, 2 * EMB), lambda i: (0, i, 0)),
            pl.BlockSpec((BT, L), lambda i: (i, 0)),
            pl.BlockSpec((BT, L), lambda i: (i, 0)),
            pl.BlockSpec((BT, 1), lambda i: (i, 0)),
            pl.BlockSpec((EMB, U), lambda i: (0, 0)),
            pl.BlockSpec((1, U), lambda i: (0, 0)),
            pl.BlockSpec((L, U), lambda i: (0, 0)),
            pl.BlockSpec((1, 1), lambda i: (0, 0)),
            pl.BlockSpec((1, 3), lambda i: (0, 0)),
            pl.BlockSpec((1, 1), lambda i: (0, 0)),
        ],
        out_specs=pl.BlockSpec((BT, 1), lambda i: (i, 0)),
        out_shape=jax.ShapeDtypeStruct((B, 1), jnp.float32),
    )(emb3, vals, one_bl, bias2, W_dense, b_dense, W_dnn_r, b_dnn, W_out_r, b_out)


def kernel(input_values, input_indexes, bias_indexes, one_order_table,
           two_order_table, W_dense, b_dense, W_dnn, b_dnn, W_out, b_out):
    idxb3 = input_indexes.astype(jnp.int32).reshape(NW, PER_W)
    bidx2 = bias_indexes.astype(jnp.int32).reshape(NW, BPW)
    t1_flat = jnp.concatenate(
        [one_order_table[:, 0],
         jnp.zeros((V1P - VOCAB - 1,), jnp.float32)])
    emb, onev, biasv = _sc_gather(idxb3, bidx2, two_order_table, t1_flat)
    emb3 = emb.reshape(L, B, 2 * EMB)
    one_bl = onev.reshape(B, L)
    bias2 = biasv.reshape(B, 1)
    return _tc_compute(
        emb3, input_values, one_bl, bias2,
        W_dense, b_dense.reshape(1, U), W_dnn.reshape(L, U),
        b_dnn.reshape(1, 1), W_out.reshape(1, 3), b_out.reshape(1, 1))


# speedup vs baseline: 1.1233x; 1.0136x over previous
import functools

import jax
import jax.numpy as jnp
from jax import lax
from jax.experimental import pallas as pl
from jax.experimental.pallas import tpu as pltpu
from jax.experimental.pallas import tpu_sc as plsc

B = 4096
L = 26
VOCAB = 100000
EMB = 64
U = 128

NC = 2
NS = 16
NW = NC * NS
N = B * L
PER_W = N // NW
K = 128
NCHUNK = PER_W // K
BPW = B // NW


V1P = 100096


def _sc_gather(idxb3, bidx2, table2, t1_flat):
    mesh = plsc.VectorSubcoreMesh(core_axis_name="c", subcore_axis_name="s")

    @functools.partial(
        pl.kernel,
        out_type=(
            jax.ShapeDtypeStruct((N, 2 * EMB), jnp.float32),
            jax.ShapeDtypeStruct((NW, NCHUNK, K), jnp.float32),
            jax.ShapeDtypeStruct((NW, BPW), jnp.float32),
        ),
        mesh=mesh,
        compiler_params=pltpu.CompilerParams(
            use_tc_tiling_on_sc=False, needs_layout_passes=False),
        scratch_types=[
            pltpu.VMEM((PER_W,), jnp.int32),
            pltpu.VMEM((2, K), jnp.int32),
            pltpu.VMEM((2, K, EMB), jnp.float32),
            pltpu.VMEM((NCHUNK, K), jnp.float32),
            pltpu.VMEM((BPW,), jnp.int32),
            pltpu.VMEM((BPW,), jnp.float32),
            pltpu.VMEM((V1P,), jnp.float32),
            pltpu.SemaphoreType.DMA,
            pltpu.SemaphoreType.DMA,
            pltpu.SemaphoreType.DMA,
        ],
    )
    def k(idxb_hbm, bidx_hbm, t2_hbm, t1_hbm, emb_hbm, one_hbm,
          bias_hbm, idxb_v, cidx_v, rows_v, one_v, bidx_v, brow_v, tab_v,
          sem0, sem1, semt):
        wid = lax.axis_index("s") * NC + lax.axis_index("c")
        dbase = wid * K

        pltpu.sync_copy(idxb_hbm.at[wid], idxb_v)
        pltpu.sync_copy(bidx_hbm.at[wid], bidx_v)
        tcopy = pltpu.async_copy(t1_hbm, tab_v, semt)

        def build(l, buf):
            for kk in range(K // 16):
                pos = (jnp.arange(16, dtype=jnp.int32) + (kk * 16)) * L + l
                cidx_v[buf, pl.ds(kk * 16, 16)] = plsc.load_gather(
                    idxb_v, [pos])

        def fire(l, buf, sem):
            pltpu.async_copy(t2_hbm.at[cidx_v.at[buf]], rows_v.at[buf], sem)

        def drain(buf, sem):
            pltpu.make_async_copy(
                t2_hbm.at[cidx_v.at[0]], rows_v.at[buf], sem).wait()

        def write(l, buf):
            pltpu.sync_copy(
                rows_v.at[buf],
                emb_hbm.at[pl.ds(l * B + dbase, K), pl.ds(0, EMB)])

        build(0, 0)
        fire(0, 0, sem0)

        def pair(p, carry):
            c0 = 2 * p
            build(c0 + 1, 1)
            fire(c0 + 1, 1, sem1)
            drain(0, sem0)
            write(c0, 0)

            @pl.when(c0 + 2 < NCHUNK)
            def _():
                build(c0 + 2, 0)
                fire(c0 + 2, 0, sem0)

            drain(1, sem1)
            write(c0 + 1, 1)
            return carry

        lax.fori_loop(0, NCHUNK // 2, pair, 0)

        tcopy.wait()
        for i in range(BPW // 16):
            brow_v[pl.ds(i * 16, 16)] = plsc.load_gather(
                tab_v, [bidx_v[pl.ds(i * 16, 16)]])
        pltpu.sync_copy(brow_v, bias_hbm.at[wid])

        def one_body(j, carry):
            for i in range(K // 16):
                one_v[j, pl.ds(i * 16, 16)] = plsc.load_gather(
                    tab_v, [idxb_v[pl.ds(j * K + i * 16, 16)]])
            return carry

        lax.fori_loop(0, NCHUNK, one_body, 0)
        pltpu.sync_copy(one_v, one_hbm.at[wid])

    return k(idxb3, bidx2, table2, t1_flat)


def _tc_body(emb_ref, vals_ref, one_ref, bias_ref, wd_ref, bd_ref,
             wdnn_ref, bdnn_ref, wout_ref, bout_ref, out_ref):
    bt = vals_ref.shape[0]
    wd = wd_ref[...]
    bd = bd_ref[...]
    sum_v = jnp.zeros((bt, EMB), jnp.float32)
    sum_sq = jnp.zeros((bt, EMB), jnp.float32)
    accd = jnp.zeros((bt, 1), jnp.float32)
    for l in range(L):
        e = emb_ref[l][:, 0:EMB]
        h = jnp.dot(e, wd, preferred_element_type=jnp.float32) + bd
        h = jnp.maximum(h, 0.0)
        accd += jnp.sum(h * wdnn_ref[l], axis=1, keepdims=True)
        v = e * vals_ref[:, l:l + 1]
        sum_v += v
        sum_sq += v * v
    y_fm1 = jnp.sum(vals_ref[...] * one_ref[...], axis=1, keepdims=True) + bias_ref[...]
    y_fm2 = 0.5 * jnp.sum(sum_v * sum_v - sum_sq, axis=1, keepdims=True)
    y_dnn = jnp.maximum(accd + bdnn_ref[...], 0.0)
    out_ref[...] = (y_fm1 * wout_ref[0:1, 0:1] + y_fm2 * wout_ref[0:1, 1:2]
                    + y_dnn * wout_ref[0:1, 2:3] + bout_ref[...])


BT = 1024


def _tc_compute(emb3, vals, one_bl, bias2, W_dense, b_dense, W_dnn_r, b_dnn,
                W_out_r, b_out):
    grid = (B // BT,)
    return pl.pallas_call(
        _tc_body,
        grid=grid,
        in_specs=[
            pl.BlockSpec((L, BT, 2 * EMB), lambda i: (0, i, 0)),
            pl.BlockSpec((BT, L), lambda i: (i, 0)),
            pl.BlockSpec((BT, L), lambda i: (i, 0)),
            pl.BlockSpec((BT, 1), lambda i: (i, 0)),
            pl.BlockSpec((EMB, U), lambda i: (0, 0)),
            pl.BlockSpec((1, U), lambda i: (0, 0)),
            pl.BlockSpec((L, U), lambda i: (0, 0)),
            pl.BlockSpec((1, 1), lambda i: (0, 0)),
            pl.BlockSpec((1, 3), lambda i: (0, 0)),
            pl.BlockSpec((1, 1), lambda i: (0, 0)),
        ],
        out_specs=pl.BlockSpec((BT, 1), lambda i: (i, 0)),
        out_shape=jax.ShapeDtypeStruct((B, 1), jnp.float32),
    )(emb3, vals, one_bl, bias2, W_dense, b_dense, W_dnn_r, b_dnn, W_out_r, b_out)


def kernel(input_values, input_indexes, bias_indexes, one_order_table,
           two_order_table, W_dense, b_dense, W_dnn, b_dnn, W_out, b_out):
    idxb3 = input_indexes.astype(jnp.int32).reshape(NW, PER_W)
    bidx2 = bias_indexes.astype(jnp.int32).reshape(NW, BPW)
    t1_flat = jnp.concatenate(
        [one_order_table[:, 0],
         jnp.zeros((V1P - VOCAB - 1,), jnp.float32)])
    emb, onev, biasv = _sc_gather(idxb3, bidx2, two_order_table, t1_flat)
    emb3 = emb.reshape(L, B, 2 * EMB)
    one_bl = onev.reshape(B, L)
    bias2 = biasv.reshape(B, 1)
    return _tc_compute(
        emb3, input_values, one_bl, bias2,
        W_dense, b_dense.reshape(1, U), W_dnn.reshape(L, U),
        b_dnn.reshape(1, 1), W_out.reshape(1, 3), b_out.reshape(1, 1))
